# baseline (speedup 1.0000x reference)
import functools
import jax
import jax.numpy as jnp
from jax import lax
from jax.experimental import pallas as pl
from jax.experimental.pallas import tpu as pltpu
from jax.experimental.pallas import tpu_sc as plsc

N = 100000
E = 1600000
BG = 1000
NFIX = 100
NC, NS, L = 2, 16, 16
NW = NC * NS
CH = 512
NJ = CH // 128
NSTEP = E // CH
EPS = 1e-5

_mesh = plsc.VectorSubcoreMesh(
    core_axis_name="c", subcore_axis_name="s", num_cores=NC, num_subcores=NS)
_sc_params = pltpu.CompilerParams(use_tc_tiling_on_sc=False)


def _wid():
    return lax.axis_index("s") * NC + lax.axis_index("c")


@functools.partial(
    pl.kernel, mesh=_mesh, compiler_params=_sc_params,
    out_type=jax.ShapeDtypeStruct((NC, N, 16), jnp.float32),
    scratch_types=[
        pltpu.VMEM((NJ, 128), jnp.int32),
        pltpu.VMEM((128, 16), jnp.float32),
        pltpu.VMEM((625, 16), jnp.float32),
        pltpu.VMEM_SHARED((N, 16), jnp.float32),
    ],
)
def _deg_kernel(dst2d, out, ibuf, ones_b, zbuf, acc):
    c = lax.axis_index("c")
    s = lax.axis_index("s")
    w = _wid()

    def initz(i, _):
        zbuf[i, :] = jnp.zeros((16,), jnp.float32)
        ones_b[jnp.minimum(i, 127), :] = jnp.ones((16,), jnp.float32)
        return 0
    lax.fori_loop(0, 625, initz, 0)
    r0 = s * (N // NS)
    for q in range(10):
        pltpu.sync_copy(zbuf, acc.at[pl.ds(r0 + q * 625, 625)])
    plsc.subcore_barrier()

    kmax = (NSTEP - w + NW - 1) // NW

    def step(k, _):
        t = w + k * NW
        pltpu.sync_copy(dst2d.at[pl.ds(t * NJ, NJ)], ibuf)
        for j in range(NJ):
            pltpu.sync_copy(ones_b, acc.at[ibuf.at[j]], add=True)
        return 0
    lax.fori_loop(0, kmax, step, 0)
    plsc.subcore_barrier()
    for q in range(10):
        pltpu.sync_copy(acc.at[pl.ds(r0 + q * 625, 625)], zbuf)
        pltpu.sync_copy(zbuf, out.at[c, pl.ds(r0 + q * 625, 625)])


@functools.partial(
    pl.kernel, mesh=_mesh, compiler_params=_sc_params,
    out_type=[jax.ShapeDtypeStruct((E, 32), jnp.float32),
              jax.ShapeDtypeStruct((NW, 4, 16), jnp.float32)],
    scratch_types=[
        pltpu.VMEM((2, NJ, 128), jnp.int32),
        pltpu.VMEM((2, NJ, 128), jnp.int32),
        pltpu.VMEM((2, CH, 32), jnp.float32),
        pltpu.VMEM((2, CH, 32), jnp.float32),
        pltpu.VMEM((2, CH, 32), jnp.float32),
        pltpu.VMEM((4, 16), jnp.float32),
        pltpu.SemaphoreType.DMA((2,)),
        pltpu.SemaphoreType.DMA((2,)),
    ],
)
def _pass1_kernel(u3, v3, src2d, dst2d, m_out, st_out,
                  isrc, idst, ubuf, vbuf, mbuf, sbuf, gsem, wsem):
    w = _wid()
    kmax = (NSTEP - w + NW - 1) // NW
    z16 = jnp.zeros((16,), jnp.float32)

    def fetch(k, p):
        t = w + k * NW
        pltpu.sync_copy(src2d.at[pl.ds(t * NJ, NJ)], isrc.at[p])
        pltpu.sync_copy(dst2d.at[pl.ds(t * NJ, NJ)], idst.at[p])
        for j in range(NJ):
            pltpu.async_copy(u3.at[idst.at[p, j]],
                             ubuf.at[p, pl.ds(j * 128, 128)], gsem.at[p])
            pltpu.async_copy(v3.at[isrc.at[p, j]],
                             vbuf.at[p, pl.ds(j * 128, 128)], gsem.at[p])

    def drain_g(k, p):
        pltpu.make_async_copy(
            u3.at[pl.ds(0, CH)], ubuf.at[p], gsem.at[p]).wait()
        pltpu.make_async_copy(
            v3.at[pl.ds(0, CH)], vbuf.at[p], gsem.at[p]).wait()

    def drain_w(p):
        pltpu.make_async_copy(
            m_out.at[pl.ds(0, CH)], mbuf.at[p], wsem.at[p]).wait()

    fetch(0, 0)

    def step(k, carry):
        sl, sh, ql, qh = carry
        p = lax.rem(k, 2)
        t = w + k * NW

        @pl.when(k + 1 < kmax)
        def _():
            fetch(k + 1, 1 - p)
        drain_g(k, p)

        @pl.when(k >= 2)
        def _():
            drain_w(p)

        def rows(i, cr):
            a0, a1, b0, b1 = cr
            for r in range(4):
                i4 = i * 4 + r
                mlo = ubuf[p, i4, pl.ds(0, 16)] + vbuf[p, i4, pl.ds(0, 16)]
                mhi = ubuf[p, i4, pl.ds(16, 16)] + vbuf[p, i4, pl.ds(16, 16)]
                mbuf[p, i4, pl.ds(0, 16)] = mlo
                mbuf[p, i4, pl.ds(16, 16)] = mhi
                a0 = a0 + mlo
                a1 = a1 + mhi
                b0 = b0 + mlo * mlo
                b1 = b1 + mhi * mhi
            return (a0, a1, b0, b1)
        sl, sh, ql, qh = lax.fori_loop(0, CH // 4, rows, (sl, sh, ql, qh))
        pltpu.async_copy(mbuf.at[p], m_out.at[pl.ds(t * CH, CH)], wsem.at[p])
        return (sl, sh, ql, qh)

    sl, sh, ql, qh = lax.fori_loop(0, kmax, step, (z16, z16, z16, z16))

    @pl.when(kmax >= 2)
    def _():
        drain_w(lax.rem(kmax, 2))

    @pl.when(kmax >= 1)
    def _():
        drain_w(lax.rem(kmax + 1, 2))
    sbuf[0, :] = sl
    sbuf[1, :] = sh
    sbuf[2, :] = ql
    sbuf[3, :] = qh
    pltpu.sync_copy(sbuf, st_out.at[w])


def _make_pass2(slope):
    @functools.partial(
        pl.kernel, mesh=_mesh, compiler_params=_sc_params,
        out_type=jax.ShapeDtypeStruct((NC, N, 16), jnp.float32),
        scratch_types=[
            pltpu.VMEM((2, NJ, 128), jnp.int32),
            pltpu.VMEM((2, CH, 16), jnp.float32),
            pltpu.VMEM((2, 2, 16), jnp.float32),
            pltpu.VMEM((625, 16), jnp.float32),
            pltpu.VMEM_SHARED((N, 16), jnp.float32),
            pltpu.SemaphoreType.DMA((2,)),
            pltpu.SemaphoreType.DMA((2,)),
        ],
    )
    def _pass2(m_in, dst2d, ab, out, ibuf, mbuf, abuf, zbuf, acc,
               rsem, ssem):
        c = lax.axis_index("c")
        s = lax.axis_index("s")
        pltpu.sync_copy(ab, abuf)

        def initz(i, _):
            zbuf[i, :] = jnp.zeros((16,), jnp.float32)
            return 0
        lax.fori_loop(0, 625, initz, 0)
        r0 = s * (N // NS)
        for q in range(10):
            pltpu.sync_copy(zbuf, acc.at[pl.ds(r0 + q * 625, 625)])
        plsc.subcore_barrier()

        av = abuf[0, c, :]
        bv = abuf[1, c, :]
        kmax = (NSTEP - s + NS - 1) // NS

        def fetch(k, p):
            t = s + k * NS
            pltpu.sync_copy(dst2d.at[pl.ds(t * NJ, NJ)], ibuf.at[p])
            pltpu.async_copy(
                m_in.at[pl.ds(t * CH, CH), pl.ds(c * 16, 16)],
                mbuf.at[p], rsem.at[p])

        def drain_r(p):
            pltpu.make_async_copy(
                m_in.at[pl.ds(0, CH), pl.ds(0, 16)], mbuf.at[p],
                rsem.at[p]).wait()

        def drain_s(p):
            pltpu.make_async_copy(
                m_in.at[pl.ds(0, CH), pl.ds(0, 16)], mbuf.at[p],
                ssem.at[p]).wait()

        fetch(0, 0)

        def step(k, _):
            p = lax.rem(k, 2)

            @pl.when(k + 1 < kmax)
            def _():
                @pl.when(k >= 1)
                def _():
                    drain_s(1 - p)
                fetch(k + 1, 1 - p)
            drain_r(p)

            def rows(i, _c):
                for r in range(4):
                    i4 = i * 4 + r
                    tv = mbuf[p, i4, :] * av + bv
                    mbuf[p, i4, :] = jnp.where(tv >= 0, tv, slope * tv)
                return 0
            lax.fori_loop(0, CH // 4, rows, 0)
            for j in range(NJ):
                pltpu.async_copy(mbuf.at[p, pl.ds(j * 128, 128)],
                                 acc.at[ibuf.at[p, j]], ssem.at[p],
                                 add=True)
            return 0
        lax.fori_loop(0, kmax, step, 0)

        @pl.when(kmax >= 2)
        def _():
            drain_s(lax.rem(kmax, 2))

        @pl.when(kmax >= 1)
        def _():
            drain_s(lax.rem(kmax + 1, 2))
        plsc.subcore_barrier()
        for q in range(10):
            pltpu.sync_copy(acc.at[pl.ds(r0 + q * 625, 625)], zbuf)
            pltpu.sync_copy(zbuf, out.at[c, pl.ds(r0 + q * 625, 625)])
    return _pass2


_pass2_act = _make_pass2(0.1)
_pass2_lin = _make_pass2(1.0)


@functools.partial(
    pl.kernel, mesh=_mesh, compiler_params=_sc_params,
    out_type=jax.ShapeDtypeStruct((BG, 64), jnp.float32),
    scratch_types=[
        pltpu.VMEM((NFIX, 16), jnp.float32),
        pltpu.VMEM((NFIX, 16), jnp.float32),
        pltpu.VMEM((NFIX, 16), jnp.float32),
        pltpu.VMEM((NFIX, 16), jnp.float32),
        pltpu.VMEM((1, 64), jnp.float32),
    ],
)
def _pool_kernel(s_hbm, deg_hbm, out, blo, bhi, bd0, bd1, obuf):
    w = _wid()
    kmax = (BG - w + NW - 1) // NW
    z16 = jnp.zeros((16,), jnp.float32)
    ninf = jnp.full((16,), -jnp.inf, jnp.float32)

    def graph(k, _):
        g = w + k * NW
        pltpu.sync_copy(s_hbm.at[0, pl.ds(g * NFIX, NFIX)], blo)
        pltpu.sync_copy(s_hbm.at[1, pl.ds(g * NFIX, NFIX)], bhi)
        pltpu.sync_copy(deg_hbm.at[0, pl.ds(g * NFIX, NFIX)], bd0)
        pltpu.sync_copy(deg_hbm.at[1, pl.ds(g * NFIX, NFIX)], bd1)

        def rows(i, cr):
            sl, sh, ml, mh = cr
            cnt = jnp.maximum(bd0[i, :] + bd1[i, :], 1.0)
            hl = blo[i, :] / cnt
            hh = bhi[i, :] / cnt
            return (sl + hl, sh + hh, jnp.maximum(ml, hl), jnp.maximum(mh, hh))
        sl, sh, ml, mh = lax.fori_loop(0, NFIX, rows, (z16, z16, ninf, ninf))
        obuf[0, pl.ds(0, 16)] = sl
        obuf[0, pl.ds(16, 16)] = sh
        obuf[0, pl.ds(32, 16)] = ml
        obuf[0, pl.ds(48, 16)] = mh
        pltpu.sync_copy(obuf, out.at[pl.ds(g, 1)])
        return 0
    lax.fori_loop(0, kmax, graph, 0)


@functools.partial(
    pl.kernel, mesh=_mesh, compiler_params=_sc_params,
    out_type=jax.ShapeDtypeStruct((N, 32), jnp.float32),
    scratch_types=[
        pltpu.VMEM((1, NFIX), jnp.int32),
        pltpu.VMEM((NFIX, 32), jnp.float32),
        pltpu.SemaphoreType.DMA,
    ],
)
def _perm_kernel(d3, perm2d, out, ibuf, gbuf, sem):
    w = _wid()
    kmax = (BG - w + NW - 1) // NW

    def step(k, _):
        r = w + k * NW
        pltpu.sync_copy(perm2d.at[pl.ds(r, 1)], ibuf)
        pltpu.async_copy(d3.at[ibuf.at[0]], gbuf, sem).wait()
        pltpu.sync_copy(gbuf, out.at[pl.ds(r * NFIX, NFIX)])
        return 0
    lax.fori_loop(0, kmax, step, 0)


def _leaky(x):
    return jnp.where(x >= 0, x, 0.1 * x)


def _uv0_body(x_ref, a_ref, c_ref, b_ref, u_ref, v_ref):
    t = x_ref[...]
    u_ref[...] = jnp.dot(t, a_ref[...],
                         preferred_element_type=jnp.float32, precision=lax.Precision.HIGHEST) + b_ref[...]
    v_ref[...] = jnp.dot(t, c_ref[...], preferred_element_type=jnp.float32, precision=lax.Precision.HIGHEST)


def _uv0(x, A, C, b, fin):
    R = 1000
    return pl.pallas_call(
        _uv0_body,
        grid=(N // R,),
        in_specs=[
            pl.BlockSpec((R, fin), lambda i: (i, 0)),
            pl.BlockSpec((fin, 32), lambda i: (0, 0)),
            pl.BlockSpec((fin, 32), lambda i: (0, 0)),
            pl.BlockSpec((1, 32), lambda i: (0, 0)),
        ],
        out_specs=[pl.BlockSpec((R, 32), lambda i: (i, 0)),
                   pl.BlockSpec((R, 32), lambda i: (i, 0))],
        out_shape=[jax.ShapeDtypeStruct((N, 32), jnp.float32),
                   jax.ShapeDtypeStruct((N, 32), jnp.float32)],
    )(x, A, C, b)


def _uvh_body(lo_ref, hi_ref, d0_ref, d1_ref, a_ref, c_ref, b_ref,
              u_ref, v_ref):
    cnt = jnp.maximum(d0_ref[...] + d1_ref[...], 1.0)
    h = jnp.concatenate([lo_ref[...], hi_ref[...]], axis=1) / cnt
    u_ref[...] = jnp.dot(h, a_ref[...],
                         preferred_element_type=jnp.float32, precision=lax.Precision.HIGHEST) + b_ref[...]
    v_ref[...] = jnp.dot(h, c_ref[...], preferred_element_type=jnp.float32, precision=lax.Precision.HIGHEST)


def _uvh(lo, hi, d0, d1, A, C, b):
    R = 1000
    return pl.pallas_call(
        _uvh_body,
        grid=(N // R,),
        in_specs=[
            pl.BlockSpec((R, 16), lambda i: (i, 0)),
            pl.BlockSpec((R, 16), lambda i: (i, 0)),
            pl.BlockSpec((R, 1), lambda i: (i, 0)),
            pl.BlockSpec((R, 1), lambda i: (i, 0)),
            pl.BlockSpec((32, 32), lambda i: (0, 0)),
            pl.BlockSpec((32, 32), lambda i: (0, 0)),
            pl.BlockSpec((1, 32), lambda i: (0, 0)),
        ],
        out_specs=[pl.BlockSpec((R, 32), lambda i: (i, 0)),
                   pl.BlockSpec((R, 32), lambda i: (i, 0))],
        out_shape=[jax.ShapeDtypeStruct((N, 32), jnp.float32),
                   jax.ShapeDtypeStruct((N, 32), jnp.float32)],
    )(lo, hi, d0, d1, A, C, b)


def _pi_wrap(t):
    pi = 3.14159265358979323846
    t = jnp.where(t >= pi, t - 2 * pi, t)
    t = jnp.where(t < -pi, t + 2 * pi, t)
    return t


def _middle_body(pool_ref, met_ref, wm_ref, bm_ref, w1_ref, b1_ref,
                 w2_ref, b2_ref, wd1_ref, bd1_ref, wdm_ref, bdm_ref,
                 wd2_ref, bd2_ref, z_ref, xm_ref, d_ref):
    p = pool_ref[...]
    hmean = p[:, :32] / float(NFIX)
    hmax = p[:, 32:]
    hm = _leaky(jnp.dot(met_ref[...], wm_ref[...],
                        preferred_element_type=jnp.float32, precision=lax.Precision.HIGHEST) + bm_ref[...])
    g = _leaky(jnp.dot(jnp.concatenate([hmean, hmax], axis=1), w1_ref[...],
                       preferred_element_type=jnp.float32, precision=lax.Precision.HIGHEST) + b1_ref[...])
    z = jnp.dot(jnp.concatenate([hm, g], axis=1), w2_ref[...],
                preferred_element_type=jnp.float32, precision=lax.Precision.HIGHEST) + b2_ref[...]
    z_ref[...] = z
    e = _leaky(jnp.dot(z, wd1_ref[...],
                       preferred_element_type=jnp.float32, precision=lax.Precision.HIGHEST) + bd1_ref[...])
    xm0 = jnp.dot(e[:, :8], wdm_ref[...],
                  preferred_element_type=jnp.float32, precision=lax.Precision.HIGHEST) + bdm_ref[...]
    xm_ref[...] = jnp.concatenate(
        [jnp.maximum(xm0[:, 0:1], 0.0), _pi_wrap(xm0[:, 1:2])], axis=1)
    d_ref[...] = _leaky(jnp.dot(e[:, 8:], wd2_ref[...],
                                preferred_element_type=jnp.float32, precision=lax.Precision.HIGHEST)
                        + bd2_ref[...])


def _middle(pool, x_met, wm, bm, w1, b1, w2, b2, wd1, bd1, wdm, bdm, wd2, bd2):
    G = 8
    return pl.pallas_call(
        _middle_body,
        grid=(BG // G,),
        in_specs=[
            pl.BlockSpec((G, 64), lambda i: (i, 0)),
            pl.BlockSpec((G, 2), lambda i: (i, 0)),
            pl.BlockSpec((2, 8), lambda i: (0, 0)),
            pl.BlockSpec((1, 8), lambda i: (0, 0)),
            pl.BlockSpec((64, 32), lambda i: (0, 0)),
            pl.BlockSpec((1, 32), lambda i: (0, 0)),
            pl.BlockSpec((40, 16), lambda i: (0, 0)),
            pl.BlockSpec((1, 16), lambda i: (0, 0)),
            pl.BlockSpec((16, 40), lambda i: (0, 0)),
            pl.BlockSpec((1, 40), lambda i: (0, 0)),
            pl.BlockSpec((8, 2), lambda i: (0, 0)),
            pl.BlockSpec((1, 2), lambda i: (0, 0)),
            pl.BlockSpec((32, 3200), lambda i: (0, 0)),
            pl.BlockSpec((1, 3200), lambda i: (0, 0)),
        ],
        out_specs=[pl.BlockSpec((G, 16), lambda i: (i, 0)),
                   pl.BlockSpec((G, 2), lambda i: (i, 0)),
                   pl.BlockSpec((G, 3200), lambda i: (i, 0))],
        out_shape=[jax.ShapeDtypeStruct((BG, 16), jnp.float32),
                   jax.ShapeDtypeStruct((BG, 2), jnp.float32),
                   jax.ShapeDtypeStruct((BG, 3200), jnp.float32)],
    )(pool, x_met, wm, bm, w1, b1, w2, b2, wd1, bd1, wdm, bdm, wd2, bd2)


def _final_body(lo_ref, d0_ref, d1_ref, o_ref):
    cnt = jnp.maximum(d0_ref[...] + d1_ref[...], 1.0)
    d = lo_ref[...] / cnt
    logits = d[:, :4]
    mx = jnp.max(logits, axis=1, keepdims=True)
    t = logits - mx
    lse = jnp.log(jnp.sum(jnp.exp(t), axis=1, keepdims=True))
    x_cat = t - lse
    x_ep = jnp.maximum(d[:, 4:6], 0.0)
    x_eta = 6.0 * jnp.tanh(d[:, 6:7])
    x_phi = 7.0 * jnp.tanh(d[:, 7:8])
    o_ref[...] = jnp.concatenate([x_cat, x_ep, x_eta, x_phi], axis=1)


def _final(lo, d0, d1):
    R = 1000
    return pl.pallas_call(
        _final_body,
        grid=(N // R,),
        in_specs=[
            pl.BlockSpec((R, 16), lambda i: (i, 0)),
            pl.BlockSpec((R, 1), lambda i: (i, 0)),
            pl.BlockSpec((R, 1), lambda i: (i, 0)),
        ],
        out_specs=pl.BlockSpec((R, 8), lambda i: (i, 0)),
        out_shape=jax.ShapeDtypeStruct((N, 8), jnp.float32),
    )(lo, d0, d1)


def _split_w(W, fin, hout):
    W1 = W[:, :fin]
    W2 = W[:, fin:]
    A = (W1 - W2).T
    C = W2.T
    if hout < 32:
        A = jnp.pad(A, ((0, 0), (0, 32 - hout)))
        C = jnp.pad(C, ((0, 0), (0, 32 - hout)))
    return A, C


def _bn_ab(stats, g, be):
    P = jnp.sum(stats, axis=0)
    s32 = jnp.concatenate([P[0], P[1]])
    q32 = jnp.concatenate([P[2], P[3]])
    mu = s32 / float(E)
    var = q32 / float(E) - mu * mu
    if g.shape[0] < 32:
        g = jnp.concatenate([g, jnp.ones((32 - g.shape[0],), jnp.float32)])
        be = jnp.concatenate([be, jnp.zeros((32 - be.shape[0],), jnp.float32)])
    alpha = g * lax.rsqrt(var + EPS)
    beta = be - mu * alpha
    return jnp.stack([alpha.reshape(2, 16), beta.reshape(2, 16)], axis=0)


def _conv(u, v, src2d, dst2d, p, pre, hout, slope):
    m, stats = _pass1_kernel(u, v, src2d, dst2d)
    ab = _bn_ab(stats, p[pre + '_g'], p[pre + '_be'])
    p2 = _pass2_act if slope == 0.1 else _pass2_lin
    return p2(m, dst2d, ab)


def kernel(x, x_met, edge_index, batch, params):
    p = params
    src2d = edge_index[0].reshape(E // 128, 128)
    dst2d = edge_index[1].reshape(E // 128, 128)

    deg = _deg_kernel(dst2d)
    d0 = deg[0, :, 0:1]
    d1 = deg[1, :, 0:1]

    xpad = jnp.pad(x, ((0, 0), (0, 3)))
    A0, C0 = _split_w(p['ec0_W'], 5, 32)
    A0 = jnp.pad(A0, ((0, 3), (0, 0)))
    C0 = jnp.pad(C0, ((0, 3), (0, 0)))
    u0, v0 = _uv0(xpad, A0, C0, p['ec0_b'].reshape(1, 32), 8)
    s0 = _conv(u0, v0, src2d, dst2d, p, 'ec0', 32, 0.1)

    A1, C1 = _split_w(p['ec1_W'], 32, 32)
    u1, v1 = _uvh(s0[0], s0[1], d0, d1,
                  A1, C1, p['ec1_b'].reshape(1, 32))
    s1 = _conv(u1, v1, src2d, dst2d, p, 'ec1', 32, 0.1)

    pool = _pool_kernel(s1, deg)
    z, xm, d = _middle(
        pool, x_met,
        p['enc_met_fc1_W'].T, p['enc_met_fc1_b'].reshape(1, 8),
        p['enc_fc1_W'].T, p['enc_fc1_b'].reshape(1, 32),
        p['enc_fc2_W'].T, p['enc_fc2_b'].reshape(1, 16),
        p['dec_fc1_W'].T, p['dec_fc1_b'].reshape(1, 40),
        p['dec_met_fc1_W'].T, p['dec_met_fc1_b'].reshape(1, 2),
        p['dec_fc2_W'].T, p['dec_fc2_b'].reshape(1, 3200))

    idx = jax.random.randint(jax.random.key(42), (BG, NFIX), 0, NFIX)
    perm2d = (idx + NFIX * jnp.arange(BG, dtype=idx.dtype)[:, None]
              ).astype(jnp.int32)
    d3 = d.reshape(N, 32)
    dperm = _perm_kernel(d3, perm2d)

    Ad0, Cd0 = _split_w(p['dc0_W'], 32, 32)
    ud0, vd0 = _uv0(dperm, Ad0, Cd0, p['dc0_b'].reshape(1, 32), 32)
    sd0 = _conv(ud0, vd0, src2d, dst2d, p, 'dc0', 32, 0.1)

    Ad1, Cd1 = _split_w(p['dc1_W'], 32, 32)
    ud1, vd1 = _uvh(sd0[0], sd0[1], d0, d1,
                    Ad1, Cd1, p['dc1_b'].reshape(1, 32))
    sd1 = _conv(ud1, vd1, src2d, dst2d, p, 'dc1', 32, 0.1)

    Ad2, Cd2 = _split_w(p['dc2_W'], 32, 8)
    b2 = jnp.pad(p['dc2_b'], (0, 24)).reshape(1, 32)
    ud2, vd2 = _uvh(sd1[0], sd1[1], d0, d1, Ad2, Cd2, b2)
    sd2 = _conv(ud2, vd2, src2d, dst2d, p, 'dc2', 8, 1.0)

    x_final = _final(sd2[0], d0, d1)
    return (x_final, xm, z)


# speedup vs baseline: 1.5926x; 1.5926x over previous
import functools
import jax
import jax.numpy as jnp
from jax import lax
from jax.experimental import pallas as pl
from jax.experimental.pallas import tpu as pltpu
from jax.experimental.pallas import tpu_sc as plsc

N = 100000
E = 1600000
BG = 1000
NFIX = 100
NC, NS, L = 2, 16, 16
NW = NC * NS
CH = 512
NJ = CH // 128
NSTEP = E // CH
EPS = 1e-5

_mesh = plsc.VectorSubcoreMesh(
    core_axis_name="c", subcore_axis_name="s", num_cores=NC, num_subcores=NS)
_sc_params = pltpu.CompilerParams(use_tc_tiling_on_sc=False)


def _wid():
    return lax.axis_index("s") * NC + lax.axis_index("c")


@functools.partial(
    pl.kernel, mesh=_mesh, compiler_params=_sc_params,
    out_type=jax.ShapeDtypeStruct((NC, N, 16), jnp.float32),
    scratch_types=[
        pltpu.VMEM((NJ, 128), jnp.int32),
        pltpu.VMEM((128, 16), jnp.float32),
        pltpu.VMEM((625, 16), jnp.float32),
        pltpu.VMEM_SHARED((N, 16), jnp.float32),
    ],
)
def _deg_kernel(dst2d, out, ibuf, ones_b, zbuf, acc):
    c = lax.axis_index("c")
    s = lax.axis_index("s")
    w = _wid()

    def initz(i, _):
        zbuf[i, :] = jnp.zeros((16,), jnp.float32)
        ones_b[jnp.minimum(i, 127), :] = jnp.ones((16,), jnp.float32)
        return 0
    lax.fori_loop(0, 625, initz, 0)
    r0 = s * (N // NS)
    for q in range(10):
        pltpu.sync_copy(zbuf, acc.at[pl.ds(r0 + q * 625, 625)])
    plsc.subcore_barrier()

    kmax = (NSTEP - w + NW - 1) // NW

    def step(k, _):
        t = w + k * NW
        pltpu.sync_copy(dst2d.at[pl.ds(t * NJ, NJ)], ibuf)
        for j in range(NJ):
            pltpu.sync_copy(ones_b, acc.at[ibuf.at[j]], add=True)
        return 0
    lax.fori_loop(0, kmax, step, 0)
    plsc.subcore_barrier()
    for q in range(10):
        pltpu.sync_copy(acc.at[pl.ds(r0 + q * 625, 625)], zbuf)
        pltpu.sync_copy(zbuf, out.at[c, pl.ds(r0 + q * 625, 625)])


@functools.partial(
    pl.kernel, mesh=_mesh, compiler_params=_sc_params,
    out_type=[jax.ShapeDtypeStruct((E, 32), jnp.float32),
              jax.ShapeDtypeStruct((NW, 4, 16), jnp.float32)],
    scratch_types=[
        pltpu.VMEM((NJ, 128), jnp.int32),
        pltpu.VMEM((NJ, 128), jnp.int32),
        pltpu.VMEM((NJ, 128), jnp.int32),
        pltpu.VMEM((NJ, 128), jnp.int32),
        pltpu.VMEM((CH, 32), jnp.float32),
        pltpu.VMEM((CH, 32), jnp.float32),
        pltpu.VMEM((CH, 32), jnp.float32),
        pltpu.VMEM((CH, 32), jnp.float32),
        pltpu.VMEM((CH, 32), jnp.float32),
        pltpu.VMEM((4, 16), jnp.float32),
        pltpu.SemaphoreType.DMA,
        pltpu.SemaphoreType.DMA,
    ],
)
def _pass1_kernel(u3, v3, src2d, dst2d, m_out, st_out,
                  isrc0, idst0, isrc1, idst1, ubuf0, vbuf0, ubuf1, vbuf1,
                  mbuf, sbuf, sem0, sem1):
    w = _wid()
    kmax = (NSTEP - w + NW - 1) // NW
    z16 = jnp.zeros((16,), jnp.float32)
    bufs = ((isrc0, idst0, ubuf0, vbuf0, sem0),
            (isrc1, idst1, ubuf1, vbuf1, sem1))

    def fetch(k, p):
        isrc, idst, ubuf, vbuf, sem = bufs[p]
        t = jnp.minimum(w + k * NW, NSTEP - 1)
        pltpu.sync_copy(src2d.at[pl.ds(t * NJ, NJ)], isrc)
        pltpu.sync_copy(dst2d.at[pl.ds(t * NJ, NJ)], idst)
        for j in range(NJ):
            pltpu.async_copy(u3.at[idst.at[j]],
                             ubuf.at[pl.ds(j * 128, 128)], sem)
            pltpu.async_copy(v3.at[isrc.at[j]],
                             vbuf.at[pl.ds(j * 128, 128)], sem)

    def halfstep(k, p, carry):
        _, _, ubuf, vbuf, sem = bufs[p]
        fetch(k + 1, 1 - p)
        pltpu.make_async_copy(u3.at[pl.ds(0, CH)], ubuf, sem).wait()
        pltpu.make_async_copy(v3.at[pl.ds(0, CH)], vbuf, sem).wait()

        def rows(i, cr):
            a0, a1, b0, b1 = cr
            for r in range(4):
                i4 = i * 4 + r
                mlo = ubuf[i4, pl.ds(0, 16)] + vbuf[i4, pl.ds(0, 16)]
                mhi = ubuf[i4, pl.ds(16, 16)] + vbuf[i4, pl.ds(16, 16)]
                mbuf[i4, pl.ds(0, 16)] = mlo
                mbuf[i4, pl.ds(16, 16)] = mhi
                a0 = a0 + mlo
                a1 = a1 + mhi
                b0 = b0 + mlo * mlo
                b1 = b1 + mhi * mhi
            return (a0, a1, b0, b1)
        carry = lax.fori_loop(0, CH // 4, rows, carry)
        t = w + k * NW
        pltpu.sync_copy(mbuf, m_out.at[pl.ds(t * CH, CH)])
        return carry

    fetch(0, 0)
    carry = halfstep(0, 0, (z16, z16, z16, z16))
    carry = halfstep(1, 1, carry)

    def pair(q, cr):
        cr = halfstep(2 * q, 0, cr)
        cr = halfstep(2 * q + 1, 1, cr)
        return cr
    carry = lax.fori_loop(1, kmax // 2, pair, carry)

    @pl.when(lax.rem(kmax, 2) == 1)
    def _():
        cr = halfstep(kmax - 1, 0, carry)
        sbuf[0, :] = cr[0]
        sbuf[1, :] = cr[1]
        sbuf[2, :] = cr[2]
        sbuf[3, :] = cr[3]

    @pl.when(lax.rem(kmax, 2) == 0)
    def _():
        sbuf[0, :] = carry[0]
        sbuf[1, :] = carry[1]
        sbuf[2, :] = carry[2]
        sbuf[3, :] = carry[3]

    @pl.when(lax.rem(kmax, 2) == 0)
    def _():
        pltpu.make_async_copy(u3.at[pl.ds(0, CH)], ubuf0, sem0).wait()
        pltpu.make_async_copy(v3.at[pl.ds(0, CH)], vbuf0, sem0).wait()

    @pl.when(lax.rem(kmax, 2) == 1)
    def _():
        pltpu.make_async_copy(u3.at[pl.ds(0, CH)], ubuf1, sem1).wait()
        pltpu.make_async_copy(v3.at[pl.ds(0, CH)], vbuf1, sem1).wait()
    pltpu.sync_copy(sbuf, st_out.at[w])


def _make_pass2(slope):
    @functools.partial(
        pl.kernel, mesh=_mesh, compiler_params=_sc_params,
        out_type=jax.ShapeDtypeStruct((NC, N, 16), jnp.float32),
        scratch_types=[
            pltpu.VMEM((NJ, 128), jnp.int32),
            pltpu.VMEM((NJ, 128), jnp.int32),
            pltpu.VMEM((CH, 16), jnp.float32),
            pltpu.VMEM((CH, 16), jnp.float32),
            pltpu.VMEM((2, 2, 16), jnp.float32),
            pltpu.VMEM((625, 16), jnp.float32),
            pltpu.VMEM_SHARED((N, 16), jnp.float32),
            pltpu.SemaphoreType.DMA,
            pltpu.SemaphoreType.DMA,
        ],
    )
    def _pass2(m_in, dst2d, ab, out, ibuf0, ibuf1, mbuf0, mbuf1,
               abuf, zbuf, acc, sem0, sem1):
        c = lax.axis_index("c")
        s = lax.axis_index("s")
        pltpu.sync_copy(ab, abuf)

        def initz(i, _):
            zbuf[i, :] = jnp.zeros((16,), jnp.float32)
            return 0
        lax.fori_loop(0, 625, initz, 0)
        r0 = s * (N // NS)
        for q in range(10):
            pltpu.sync_copy(zbuf, acc.at[pl.ds(r0 + q * 625, 625)])
        plsc.subcore_barrier()

        av = abuf[0, c, :]
        bv = abuf[1, c, :]
        kmax = (NSTEP - s + NS - 1) // NS
        bufs = ((ibuf0, mbuf0, sem0), (ibuf1, mbuf1, sem1))

        def fetch(k, p):
            ibuf, mbuf, sem = bufs[p]
            t = jnp.minimum(s + k * NS, NSTEP - 1)
            pltpu.sync_copy(dst2d.at[pl.ds(t * NJ, NJ)], ibuf)
            pltpu.async_copy(
                m_in.at[pl.ds(t * CH, CH), pl.ds(c * 16, 16)], mbuf, sem)

        def halfstep(k, p):
            ibuf, mbuf, sem = bufs[p]
            fetch(k + 1, 1 - p)
            pltpu.make_async_copy(
                m_in.at[pl.ds(0, CH), pl.ds(0, 16)], mbuf, sem).wait()

            def rows(i, _c):
                for r in range(4):
                    i4 = i * 4 + r
                    tv = mbuf[i4, :] * av + bv
                    mbuf[i4, :] = jnp.where(tv >= 0, tv, slope * tv)
                return 0
            lax.fori_loop(0, CH // 4, rows, 0)
            for j in range(NJ):
                pltpu.sync_copy(mbuf.at[pl.ds(j * 128, 128)],
                                acc.at[ibuf.at[j]], add=True)

        fetch(0, 0)
        halfstep(0, 0)
        halfstep(1, 1)

        def pair(q, _):
            halfstep(2 * q, 0)
            halfstep(2 * q + 1, 1)
            return 0
        lax.fori_loop(1, kmax // 2, pair, 0)

        @pl.when(lax.rem(kmax, 2) == 1)
        def _():
            halfstep(kmax - 1, 0)

        @pl.when(lax.rem(kmax, 2) == 0)
        def _():
            pltpu.make_async_copy(
                m_in.at[pl.ds(0, CH), pl.ds(0, 16)], mbuf0, sem0).wait()

        @pl.when(lax.rem(kmax, 2) == 1)
        def _():
            pltpu.make_async_copy(
                m_in.at[pl.ds(0, CH), pl.ds(0, 16)], mbuf1, sem1).wait()
        plsc.subcore_barrier()
        for q in range(10):
            pltpu.sync_copy(acc.at[pl.ds(r0 + q * 625, 625)], zbuf)
            pltpu.sync_copy(zbuf, out.at[c, pl.ds(r0 + q * 625, 625)])
    return _pass2


_pass2_act = _make_pass2(0.1)
_pass2_lin = _make_pass2(1.0)


@functools.partial(
    pl.kernel, mesh=_mesh, compiler_params=_sc_params,
    out_type=jax.ShapeDtypeStruct((BG, 64), jnp.float32),
    scratch_types=[
        pltpu.VMEM((NFIX, 16), jnp.float32),
        pltpu.VMEM((NFIX, 16), jnp.float32),
        pltpu.VMEM((NFIX, 16), jnp.float32),
        pltpu.VMEM((NFIX, 16), jnp.float32),
        pltpu.VMEM((1, 64), jnp.float32),
    ],
)
def _pool_kernel(s_hbm, deg_hbm, out, blo, bhi, bd0, bd1, obuf):
    w = _wid()
    kmax = (BG - w + NW - 1) // NW
    z16 = jnp.zeros((16,), jnp.float32)
    ninf = jnp.full((16,), -jnp.inf, jnp.float32)

    def graph(k, _):
        g = w + k * NW
        pltpu.sync_copy(s_hbm.at[0, pl.ds(g * NFIX, NFIX)], blo)
        pltpu.sync_copy(s_hbm.at[1, pl.ds(g * NFIX, NFIX)], bhi)
        pltpu.sync_copy(deg_hbm.at[0, pl.ds(g * NFIX, NFIX)], bd0)
        pltpu.sync_copy(deg_hbm.at[1, pl.ds(g * NFIX, NFIX)], bd1)

        def rows(i, cr):
            sl, sh, ml, mh = cr
            cnt = jnp.maximum(bd0[i, :] + bd1[i, :], 1.0)
            hl = blo[i, :] / cnt
            hh = bhi[i, :] / cnt
            return (sl + hl, sh + hh, jnp.maximum(ml, hl), jnp.maximum(mh, hh))
        sl, sh, ml, mh = lax.fori_loop(0, NFIX, rows, (z16, z16, ninf, ninf))
        obuf[0, pl.ds(0, 16)] = sl
        obuf[0, pl.ds(16, 16)] = sh
        obuf[0, pl.ds(32, 16)] = ml
        obuf[0, pl.ds(48, 16)] = mh
        pltpu.sync_copy(obuf, out.at[pl.ds(g, 1)])
        return 0
    lax.fori_loop(0, kmax, graph, 0)


@functools.partial(
    pl.kernel, mesh=_mesh, compiler_params=_sc_params,
    out_type=jax.ShapeDtypeStruct((N, 32), jnp.float32),
    scratch_types=[
        pltpu.VMEM((1, NFIX), jnp.int32),
        pltpu.VMEM((NFIX, 32), jnp.float32),
        pltpu.SemaphoreType.DMA,
    ],
)
def _perm_kernel(d3, perm2d, out, ibuf, gbuf, sem):
    w = _wid()
    kmax = (BG - w + NW - 1) // NW

    def step(k, _):
        r = w + k * NW
        pltpu.sync_copy(perm2d.at[pl.ds(r, 1)], ibuf)
        pltpu.async_copy(d3.at[ibuf.at[0]], gbuf, sem).wait()
        pltpu.sync_copy(gbuf, out.at[pl.ds(r * NFIX, NFIX)])
        return 0
    lax.fori_loop(0, kmax, step, 0)


def _leaky(x):
    return jnp.where(x >= 0, x, 0.1 * x)


def _uv0_body(x_ref, a_ref, c_ref, b_ref, u_ref, v_ref):
    t = x_ref[...]
    u_ref[...] = jnp.dot(t, a_ref[...],
                         preferred_element_type=jnp.float32, precision=lax.Precision.HIGHEST) + b_ref[...]
    v_ref[...] = jnp.dot(t, c_ref[...], preferred_element_type=jnp.float32, precision=lax.Precision.HIGHEST)


def _uv0(x, A, C, b, fin):
    R = 1000
    return pl.pallas_call(
        _uv0_body,
        grid=(N // R,),
        in_specs=[
            pl.BlockSpec((R, fin), lambda i: (i, 0)),
            pl.BlockSpec((fin, 32), lambda i: (0, 0)),
            pl.BlockSpec((fin, 32), lambda i: (0, 0)),
            pl.BlockSpec((1, 32), lambda i: (0, 0)),
        ],
        out_specs=[pl.BlockSpec((R, 32), lambda i: (i, 0)),
                   pl.BlockSpec((R, 32), lambda i: (i, 0))],
        out_shape=[jax.ShapeDtypeStruct((N, 32), jnp.float32),
                   jax.ShapeDtypeStruct((N, 32), jnp.float32)],
    )(x, A, C, b)


def _uvh_body(lo_ref, hi_ref, d0_ref, d1_ref, a_ref, c_ref, b_ref,
              u_ref, v_ref):
    cnt = jnp.maximum(d0_ref[...] + d1_ref[...], 1.0)
    h = jnp.concatenate([lo_ref[...], hi_ref[...]], axis=1) / cnt
    u_ref[...] = jnp.dot(h, a_ref[...],
                         preferred_element_type=jnp.float32, precision=lax.Precision.HIGHEST) + b_ref[...]
    v_ref[...] = jnp.dot(h, c_ref[...], preferred_element_type=jnp.float32, precision=lax.Precision.HIGHEST)


def _uvh(lo, hi, d0, d1, A, C, b):
    R = 1000
    return pl.pallas_call(
        _uvh_body,
        grid=(N // R,),
        in_specs=[
            pl.BlockSpec((R, 16), lambda i: (i, 0)),
            pl.BlockSpec((R, 16), lambda i: (i, 0)),
            pl.BlockSpec((R, 1), lambda i: (i, 0)),
            pl.BlockSpec((R, 1), lambda i: (i, 0)),
            pl.BlockSpec((32, 32), lambda i: (0, 0)),
            pl.BlockSpec((32, 32), lambda i: (0, 0)),
            pl.BlockSpec((1, 32), lambda i: (0, 0)),
        ],
        out_specs=[pl.BlockSpec((R, 32), lambda i: (i, 0)),
                   pl.BlockSpec((R, 32), lambda i: (i, 0))],
        out_shape=[jax.ShapeDtypeStruct((N, 32), jnp.float32),
                   jax.ShapeDtypeStruct((N, 32), jnp.float32)],
    )(lo, hi, d0, d1, A, C, b)


def _pi_wrap(t):
    pi = 3.14159265358979323846
    t = jnp.where(t >= pi, t - 2 * pi, t)
    t = jnp.where(t < -pi, t + 2 * pi, t)
    return t


def _middle_body(pool_ref, met_ref, wm_ref, bm_ref, w1_ref, b1_ref,
                 w2_ref, b2_ref, wd1_ref, bd1_ref, wdm_ref, bdm_ref,
                 wd2_ref, bd2_ref, z_ref, xm_ref, d_ref):
    p = pool_ref[...]
    hmean = p[:, :32] / float(NFIX)
    hmax = p[:, 32:]
    hm = _leaky(jnp.dot(met_ref[...], wm_ref[...],
                        preferred_element_type=jnp.float32, precision=lax.Precision.HIGHEST) + bm_ref[...])
    g = _leaky(jnp.dot(jnp.concatenate([hmean, hmax], axis=1), w1_ref[...],
                       preferred_element_type=jnp.float32, precision=lax.Precision.HIGHEST) + b1_ref[...])
    z = jnp.dot(jnp.concatenate([hm, g], axis=1), w2_ref[...],
                preferred_element_type=jnp.float32, precision=lax.Precision.HIGHEST) + b2_ref[...]
    z_ref[...] = z
    e = _leaky(jnp.dot(z, wd1_ref[...],
                       preferred_element_type=jnp.float32, precision=lax.Precision.HIGHEST) + bd1_ref[...])
    xm0 = jnp.dot(e[:, :8], wdm_ref[...],
                  preferred_element_type=jnp.float32, precision=lax.Precision.HIGHEST) + bdm_ref[...]
    xm_ref[...] = jnp.concatenate(
        [jnp.maximum(xm0[:, 0:1], 0.0), _pi_wrap(xm0[:, 1:2])], axis=1)
    d_ref[...] = _leaky(jnp.dot(e[:, 8:], wd2_ref[...],
                                preferred_element_type=jnp.float32, precision=lax.Precision.HIGHEST)
                        + bd2_ref[...])


def _middle(pool, x_met, wm, bm, w1, b1, w2, b2, wd1, bd1, wdm, bdm, wd2, bd2):
    G = 8
    return pl.pallas_call(
        _middle_body,
        grid=(BG // G,),
        in_specs=[
            pl.BlockSpec((G, 64), lambda i: (i, 0)),
            pl.BlockSpec((G, 2), lambda i: (i, 0)),
            pl.BlockSpec((2, 8), lambda i: (0, 0)),
            pl.BlockSpec((1, 8), lambda i: (0, 0)),
            pl.BlockSpec((64, 32), lambda i: (0, 0)),
            pl.BlockSpec((1, 32), lambda i: (0, 0)),
            pl.BlockSpec((40, 16), lambda i: (0, 0)),
            pl.BlockSpec((1, 16), lambda i: (0, 0)),
            pl.BlockSpec((16, 40), lambda i: (0, 0)),
            pl.BlockSpec((1, 40), lambda i: (0, 0)),
            pl.BlockSpec((8, 2), lambda i: (0, 0)),
            pl.BlockSpec((1, 2), lambda i: (0, 0)),
            pl.BlockSpec((32, 3200), lambda i: (0, 0)),
            pl.BlockSpec((1, 3200), lambda i: (0, 0)),
        ],
        out_specs=[pl.BlockSpec((G, 16), lambda i: (i, 0)),
                   pl.BlockSpec((G, 2), lambda i: (i, 0)),
                   pl.BlockSpec((G, 3200), lambda i: (i, 0))],
        out_shape=[jax.ShapeDtypeStruct((BG, 16), jnp.float32),
                   jax.ShapeDtypeStruct((BG, 2), jnp.float32),
                   jax.ShapeDtypeStruct((BG, 3200), jnp.float32)],
    )(pool, x_met, wm, bm, w1, b1, w2, b2, wd1, bd1, wdm, bdm, wd2, bd2)


def _final_body(lo_ref, d0_ref, d1_ref, o_ref):
    cnt = jnp.maximum(d0_ref[...] + d1_ref[...], 1.0)
    d = lo_ref[...] / cnt
    logits = d[:, :4]
    mx = jnp.max(logits, axis=1, keepdims=True)
    t = logits - mx
    lse = jnp.log(jnp.sum(jnp.exp(t), axis=1, keepdims=True))
    x_cat = t - lse
    x_ep = jnp.maximum(d[:, 4:6], 0.0)
    x_eta = 6.0 * jnp.tanh(d[:, 6:7])
    x_phi = 7.0 * jnp.tanh(d[:, 7:8])
    o_ref[...] = jnp.concatenate([x_cat, x_ep, x_eta, x_phi], axis=1)


def _final(lo, d0, d1):
    R = 1000
    return pl.pallas_call(
        _final_body,
        grid=(N // R,),
        in_specs=[
            pl.BlockSpec((R, 16), lambda i: (i, 0)),
            pl.BlockSpec((R, 1), lambda i: (i, 0)),
            pl.BlockSpec((R, 1), lambda i: (i, 0)),
        ],
        out_specs=pl.BlockSpec((R, 8), lambda i: (i, 0)),
        out_shape=jax.ShapeDtypeStruct((N, 8), jnp.float32),
    )(lo, d0, d1)


def _split_w(W, fin, hout):
    W1 = W[:, :fin]
    W2 = W[:, fin:]
    A = (W1 - W2).T
    C = W2.T
    if hout < 32:
        A = jnp.pad(A, ((0, 0), (0, 32 - hout)))
        C = jnp.pad(C, ((0, 0), (0, 32 - hout)))
    return A, C


def _bn_ab(stats, g, be):
    P = jnp.sum(stats, axis=0)
    s32 = jnp.concatenate([P[0], P[1]])
    q32 = jnp.concatenate([P[2], P[3]])
    mu = s32 / float(E)
    var = q32 / float(E) - mu * mu
    if g.shape[0] < 32:
        g = jnp.concatenate([g, jnp.ones((32 - g.shape[0],), jnp.float32)])
        be = jnp.concatenate([be, jnp.zeros((32 - be.shape[0],), jnp.float32)])
    alpha = g * lax.rsqrt(var + EPS)
    beta = be - mu * alpha
    return jnp.stack([alpha.reshape(2, 16), beta.reshape(2, 16)], axis=0)


def _conv(u, v, src2d, dst2d, p, pre, hout, slope):
    m, stats = _pass1_kernel(u, v, src2d, dst2d)
    ab = _bn_ab(stats, p[pre + '_g'], p[pre + '_be'])
    p2 = _pass2_act if slope == 0.1 else _pass2_lin
    return p2(m, dst2d, ab)


def kernel(x, x_met, edge_index, batch, params):
    p = params
    src2d = edge_index[0].reshape(E // 128, 128)
    dst2d = edge_index[1].reshape(E // 128, 128)

    deg = _deg_kernel(dst2d)
    d0 = deg[0, :, 0:1]
    d1 = deg[1, :, 0:1]

    xpad = jnp.pad(x, ((0, 0), (0, 3)))
    A0, C0 = _split_w(p['ec0_W'], 5, 32)
    A0 = jnp.pad(A0, ((0, 3), (0, 0)))
    C0 = jnp.pad(C0, ((0, 3), (0, 0)))
    u0, v0 = _uv0(xpad, A0, C0, p['ec0_b'].reshape(1, 32), 8)
    s0 = _conv(u0, v0, src2d, dst2d, p, 'ec0', 32, 0.1)

    A1, C1 = _split_w(p['ec1_W'], 32, 32)
    u1, v1 = _uvh(s0[0], s0[1], d0, d1,
                  A1, C1, p['ec1_b'].reshape(1, 32))
    s1 = _conv(u1, v1, src2d, dst2d, p, 'ec1', 32, 0.1)

    pool = _pool_kernel(s1, deg)
    z, xm, d = _middle(
        pool, x_met,
        p['enc_met_fc1_W'].T, p['enc_met_fc1_b'].reshape(1, 8),
        p['enc_fc1_W'].T, p['enc_fc1_b'].reshape(1, 32),
        p['enc_fc2_W'].T, p['enc_fc2_b'].reshape(1, 16),
        p['dec_fc1_W'].T, p['dec_fc1_b'].reshape(1, 40),
        p['dec_met_fc1_W'].T, p['dec_met_fc1_b'].reshape(1, 2),
        p['dec_fc2_W'].T, p['dec_fc2_b'].reshape(1, 3200))

    idx = jax.random.randint(jax.random.key(42), (BG, NFIX), 0, NFIX)
    perm2d = (idx + NFIX * jnp.arange(BG, dtype=idx.dtype)[:, None]
              ).astype(jnp.int32)
    d3 = d.reshape(N, 32)
    dperm = _perm_kernel(d3, perm2d)

    Ad0, Cd0 = _split_w(p['dc0_W'], 32, 32)
    ud0, vd0 = _uv0(dperm, Ad0, Cd0, p['dc0_b'].reshape(1, 32), 32)
    sd0 = _conv(ud0, vd0, src2d, dst2d, p, 'dc0', 32, 0.1)

    Ad1, Cd1 = _split_w(p['dc1_W'], 32, 32)
    ud1, vd1 = _uvh(sd0[0], sd0[1], d0, d1,
                    Ad1, Cd1, p['dc1_b'].reshape(1, 32))
    sd1 = _conv(ud1, vd1, src2d, dst2d, p, 'dc1', 32, 0.1)

    Ad2, Cd2 = _split_w(p['dc2_W'], 32, 8)
    b2 = jnp.pad(p['dc2_b'], (0, 24)).reshape(1, 32)
    ud2, vd2 = _uvh(sd1[0], sd1[1], d0, d1, Ad2, Cd2, b2)
    sd2 = _conv(ud2, vd2, src2d, dst2d, p, 'dc2', 8, 1.0)

    x_final = _final(sd2[0], d0, d1)
    return (x_final, xm, z)
